# baseline (speedup 1.0000x reference)
import functools

import jax
import jax.numpy as jnp
from jax import lax
from jax.experimental import pallas as pl
from jax.experimental.pallas import tpu as pltpu
from jax.experimental.pallas import tpu_sc as plsc

BATCH = 16384
NUM_ENT = 1000000
D = 64
NUM_CORES = 2
NUM_SUBCORES = 16
NW = NUM_CORES * NUM_SUBCORES
B_PER_W = BATCH // NW
W = 16
NCHUNK = B_PER_W // W

TBLK = 8192


def _transpose_block(*refs):
    ins = refs[:4]
    o = refs[4]
    row = lax.broadcasted_iota(jnp.int32, (128, 128), 0)
    col = lax.broadcasted_iota(jnp.int32, (128, 128), 1)
    eye = (row == col).astype(jnp.bfloat16)
    for tbl, t in enumerate(ins):
        x = t[...].astype(jnp.bfloat16)
        for j in range(TBLK // 128):
            tile = x[:, j * 128:(j + 1) * 128]
            c = lax.dot_general(
                eye, tile, (((1,), (1,)), ((), ())),
                preferred_element_type=jnp.float32)
            au = lax.bitcast_convert_type(c[:, :D // 2], jnp.uint32)
            bu = lax.bitcast_convert_type(c[:, D // 2:], jnp.uint32)
            o[pl.ds(j * 128, 128), pl.ds(tbl * 32, 32)] = (
                (au >> 16) | (bu & jnp.uint32(0xFFFF0000)))


def _transpose_tables(tables_t):
    grid = (NUM_ENT + TBLK - 1) // TBLK
    return pl.pallas_call(
        _transpose_block,
        grid=(grid,),
        in_specs=[pl.BlockSpec((D, TBLK), lambda i: (0, i))] * 4,
        out_specs=[pl.BlockSpec((TBLK, 128), lambda i: (i, 0))],
        out_shape=[jax.ShapeDtypeStruct((NUM_ENT, 128), jnp.uint32)],
        compiler_params=pltpu.CompilerParams(
            dimension_semantics=("parallel",)),
    )(*tables_t)[0]


def _lane_perm(x, idx):
    dnums = lax.GatherDimensionNumbers(
        offset_dims=(), collapsed_slice_dims=(0,), start_index_map=(0,))
    return lax.gather(x, idx[:, None], dnums, slice_sizes=(1,),
                      mode=lax.GatherScatterMode.PROMISE_IN_BOUNDS)


def _body(heads_hbm, rels_hbm, tails_hbm, tab_hbm,
          out_hbm, h_idx, r_idx, t_idx,
          bufs_a, bufs_b, out_v, sem_a, sem_b):
    cid = lax.axis_index("c")
    sid = lax.axis_index("s")
    wid = sid * NUM_CORES + cid
    base = wid * B_PER_W

    pltpu.sync_copy(heads_hbm.at[pl.ds(base, B_PER_W)], h_idx)
    pltpu.sync_copy(rels_hbm.at[pl.ds(base, B_PER_W)], r_idx)
    pltpu.sync_copy(tails_hbm.at[pl.ds(base, B_PER_W)], t_idx)

    iota = lax.iota(jnp.int32, 16)

    def issue(chunk, bufs, sem):
        h_buf, t_buf, r_buf = bufs
        off = pl.multiple_of(chunk * W, W)
        hvec = h_idx[pl.ds(off, W)]
        rvec = r_idx[pl.ds(off, W)]
        tvec = t_idx[pl.ds(off, W)]
        for j in range(W):
            pltpu.async_copy(tab_hbm.at[hvec[j]], h_buf.at[j], sem)
            pltpu.async_copy(tab_hbm.at[tvec[j]], t_buf.at[j], sem)
            pltpu.async_copy(tab_hbm.at[rvec[j]], r_buf.at[j], sem)

    def drain(bufs, sem):
        for buf in bufs:
            pltpu.make_async_copy(tab_hbm.at[pl.ds(0, W)], buf, sem).wait()

    hi_mask = jnp.full((16,), 0xFFFF0000, jnp.uint32)

    def compute(chunk, bufs):
        h_buf, t_buf, r_buf = bufs
        acc = jnp.zeros((16,), jnp.float32)
        for j in range(W):
            s = None
            for q in range(2):
                words = [
                    h_buf[j, pl.ds(q * 16, 16)],
                    t_buf[j, pl.ds(q * 16, 16)],
                    h_buf[j, pl.ds(32 + q * 16, 16)],
                    t_buf[j, pl.ds(32 + q * 16, 16)],
                    r_buf[j, pl.ds(64 + q * 16, 16)],
                    r_buf[j, pl.ds(96 + q * 16, 16)],
                ]
                for half in range(2):
                    if half == 0:
                        vals = [lax.bitcast_convert_type(w << 16, jnp.float32)
                                for w in words]
                    else:
                        vals = [lax.bitcast_convert_type(w & hi_mask,
                                                         jnp.float32)
                                for w in words]
                    hh, ht, th, tt, r, ri = vals
                    p = hh * r * tt + ht * ri * th
                    s = p if s is None else s + p
            for sh in (8, 4, 2, 1):
                s = s + _lane_perm(s, iota ^ sh)
            acc = jnp.where(iota == j, s, acc)
        acc = jnp.clip(acc * 0.5, -20.0, 20.0)
        out_v[pl.ds(pl.multiple_of(chunk * W, W), W)] = acc

    issue(0, bufs_a, sem_a)

    def pair_body(k, _):
        c0 = k * 2
        issue(c0 + 1, bufs_b, sem_b)
        drain(bufs_a, sem_a)
        compute(c0, bufs_a)

        @pl.when(c0 + 2 < NCHUNK)
        def _():
            issue(c0 + 2, bufs_a, sem_a)

        drain(bufs_b, sem_b)
        compute(c0 + 1, bufs_b)
        return 0

    lax.fori_loop(0, NCHUNK // 2, pair_body, 0)

    pltpu.sync_copy(out_v, out_hbm.at[pl.ds(base, B_PER_W)])


def kernel(batch, ent_h_embs, ent_t_embs, rel_embs, rel_inv_embs):
    mesh = plsc.VectorSubcoreMesh(core_axis_name="c", subcore_axis_name="s")
    k = functools.partial(
        pl.kernel,
        mesh=mesh,
        out_type=jax.ShapeDtypeStruct((BATCH,), jnp.float32),
        scratch_types=[
            pltpu.VMEM((B_PER_W,), jnp.int32),
            pltpu.VMEM((B_PER_W,), jnp.int32),
            pltpu.VMEM((B_PER_W,), jnp.int32),
            [pltpu.VMEM((W, 128), jnp.uint32) for _ in range(3)],
            [pltpu.VMEM((W, 128), jnp.uint32) for _ in range(3)],
            pltpu.VMEM((B_PER_W,), jnp.float32),
            pltpu.SemaphoreType.DMA,
            pltpu.SemaphoreType.DMA,
        ],
    )(_body)
    heads = batch[:, 0]
    rels = batch[:, 1]
    tails = batch[:, 2]
    tab = _transpose_tables(
        (ent_h_embs.T, ent_t_embs.T, rel_embs.T, rel_inv_embs.T))
    return k(heads, rels, tails, tab)


# speedup vs baseline: 2.2165x; 2.2165x over previous
import functools

import jax
import jax.numpy as jnp
from jax import lax
from jax.experimental import pallas as pl
from jax.experimental.pallas import tpu as pltpu
from jax.experimental.pallas import tpu_sc as plsc

BATCH = 16384
NUM_ENT = 1000000
D = 64
NUM_CORES = 2
NUM_SUBCORES = 16
NW = NUM_CORES * NUM_SUBCORES
B_PER_W = BATCH // NW
W = 16
NCHUNK = B_PER_W // W

TBLK = 8192


def _transpose_block(*refs):
    ins = refs[:4]
    outs = refs[4:]
    row = lax.broadcasted_iota(jnp.int32, (128, 128), 0)
    col = lax.broadcasted_iota(jnp.int32, (128, 128), 1)
    eye = (row == col).astype(jnp.bfloat16)
    for t, o in zip(ins, outs):
        x = t[...].astype(jnp.bfloat16)
        for j in range(TBLK // 128):
            tile = x[:, j * 128:(j + 1) * 128]
            c = lax.dot_general(
                eye, tile, (((1,), (1,)), ((), ())),
                preferred_element_type=jnp.float32)
            au = lax.bitcast_convert_type(c[:, :D // 2], jnp.uint32)
            bu = lax.bitcast_convert_type(c[:, D // 2:], jnp.uint32)
            o[pl.ds(j * 128, 128), :] = (
                (au >> 16) | (bu & jnp.uint32(0xFFFF0000)))


def _transpose_tables(tables_t):
    grid = (NUM_ENT + TBLK - 1) // TBLK
    return pl.pallas_call(
        _transpose_block,
        grid=(grid,),
        in_specs=[pl.BlockSpec((D, TBLK), lambda i: (0, i))] * 4,
        out_specs=[pl.BlockSpec((TBLK, D // 2), lambda i: (i, 0))] * 4,
        out_shape=[jax.ShapeDtypeStruct((NUM_ENT, D // 2), jnp.uint32)] * 4,
        compiler_params=pltpu.CompilerParams(
            dimension_semantics=("parallel",)),
    )(*tables_t)


def _lane_perm(x, idx):
    dnums = lax.GatherDimensionNumbers(
        offset_dims=(), collapsed_slice_dims=(0,), start_index_map=(0,))
    return lax.gather(x, idx[:, None], dnums, slice_sizes=(1,),
                      mode=lax.GatherScatterMode.PROMISE_IN_BOUNDS)


def _body(heads_hbm, rels_hbm, tails_hbm, eh_hbm, et_hbm, r_hbm, ri_hbm,
          out_hbm, h_idx, r_idx, t_idx,
          bufs_a, bufs_b, out_v, sem_a, sem_b):
    cid = lax.axis_index("c")
    sid = lax.axis_index("s")
    wid = sid * NUM_CORES + cid
    base = wid * B_PER_W

    pltpu.sync_copy(heads_hbm.at[pl.ds(base, B_PER_W)], h_idx)
    pltpu.sync_copy(rels_hbm.at[pl.ds(base, B_PER_W)], r_idx)
    pltpu.sync_copy(tails_hbm.at[pl.ds(base, B_PER_W)], t_idx)

    iota = lax.iota(jnp.int32, 16)

    def issue(chunk, bufs, sem):
        hh_v, ht_v, th_v, tt_v, r_v, ri_v = bufs
        off = pl.multiple_of(chunk * W, W)
        hvec = h_idx[pl.ds(off, W)]
        rvec = r_idx[pl.ds(off, W)]
        tvec = t_idx[pl.ds(off, W)]
        for j in range(W):
            hv = hvec[j]
            rv = rvec[j]
            tv = tvec[j]
            pltpu.async_copy(eh_hbm.at[hv], hh_v.at[j], sem)
            pltpu.async_copy(eh_hbm.at[tv], ht_v.at[j], sem)
            pltpu.async_copy(et_hbm.at[hv], th_v.at[j], sem)
            pltpu.async_copy(et_hbm.at[tv], tt_v.at[j], sem)
            pltpu.async_copy(r_hbm.at[rv], r_v.at[j], sem)
            pltpu.async_copy(ri_hbm.at[rv], ri_v.at[j], sem)

    def drain(bufs, sem):
        for buf in bufs:
            pltpu.make_async_copy(eh_hbm.at[pl.ds(0, W)], buf, sem).wait()

    hi_mask = jnp.full((16,), 0xFFFF0000, jnp.uint32)

    def compute(chunk, bufs):
        acc = jnp.zeros((16,), jnp.float32)
        for j in range(W):
            s = None
            for q in range(2):
                sl = pl.ds(q * 16, 16)
                words = [buf[j, sl] for buf in bufs]
                for half in range(2):
                    if half == 0:
                        vals = [lax.bitcast_convert_type(w << 16, jnp.float32)
                                for w in words]
                    else:
                        vals = [lax.bitcast_convert_type(w & hi_mask,
                                                         jnp.float32)
                                for w in words]
                    hh, ht, th, tt, r, ri = vals
                    p = hh * r * tt + ht * ri * th
                    s = p if s is None else s + p
            for sh in (8, 4, 2, 1):
                s = s + _lane_perm(s, iota ^ sh)
            acc = jnp.where(iota == j, s, acc)
        acc = jnp.clip(acc * 0.5, -20.0, 20.0)
        out_v[pl.ds(pl.multiple_of(chunk * W, W), W)] = acc

    issue(0, bufs_a, sem_a)

    def pair_body(k, _):
        c0 = k * 2
        issue(c0 + 1, bufs_b, sem_b)
        drain(bufs_a, sem_a)
        compute(c0, bufs_a)

        @pl.when(c0 + 2 < NCHUNK)
        def _():
            issue(c0 + 2, bufs_a, sem_a)

        drain(bufs_b, sem_b)
        compute(c0 + 1, bufs_b)
        return 0

    lax.fori_loop(0, NCHUNK // 2, pair_body, 0)

    pltpu.sync_copy(out_v, out_hbm.at[pl.ds(base, B_PER_W)])


def kernel(batch, ent_h_embs, ent_t_embs, rel_embs, rel_inv_embs):
    mesh = plsc.VectorSubcoreMesh(core_axis_name="c", subcore_axis_name="s")
    k = functools.partial(
        pl.kernel,
        mesh=mesh,
        out_type=jax.ShapeDtypeStruct((BATCH,), jnp.float32),
        scratch_types=[
            pltpu.VMEM((B_PER_W,), jnp.int32),
            pltpu.VMEM((B_PER_W,), jnp.int32),
            pltpu.VMEM((B_PER_W,), jnp.int32),
            [pltpu.VMEM((W, D // 2), jnp.uint32) for _ in range(6)],
            [pltpu.VMEM((W, D // 2), jnp.uint32) for _ in range(6)],
            pltpu.VMEM((B_PER_W,), jnp.float32),
            pltpu.SemaphoreType.DMA,
            pltpu.SemaphoreType.DMA,
        ],
    )(_body)
    heads = batch[:, 0]
    rels = batch[:, 1]
    tails = batch[:, 2]
    eh, et, r, ri = _transpose_tables(
        (ent_h_embs.T, ent_t_embs.T, rel_embs.T, rel_inv_embs.T))
    return k(heads, rels, tails, eh, et, r, ri)


# speedup vs baseline: 2.2227x; 1.0028x over previous
import functools

import jax
import jax.numpy as jnp
from jax import lax
from jax.experimental import pallas as pl
from jax.experimental.pallas import tpu as pltpu
from jax.experimental.pallas import tpu_sc as plsc

BATCH = 16384
NUM_ENT = 1000000
D = 64
NUM_CORES = 2
NUM_SUBCORES = 16
NW = NUM_CORES * NUM_SUBCORES
B_PER_W = BATCH // NW
W = 16
NCHUNK = B_PER_W // W

TBLK = 9216


def _transpose_block(*refs):
    ins = refs[:4]
    outs = refs[4:]
    row = lax.broadcasted_iota(jnp.int32, (128, 128), 0)
    col = lax.broadcasted_iota(jnp.int32, (128, 128), 1)
    eye = (row == col).astype(jnp.bfloat16)
    for t, o in zip(ins, outs):
        x = t[...].astype(jnp.bfloat16)
        for j in range(TBLK // 128):
            tile = x[:, j * 128:(j + 1) * 128]
            c = lax.dot_general(
                eye, tile, (((1,), (1,)), ((), ())),
                preferred_element_type=jnp.float32)
            au = lax.bitcast_convert_type(c[:, :D // 2], jnp.uint32)
            bu = lax.bitcast_convert_type(c[:, D // 2:], jnp.uint32)
            o[pl.ds(j * 128, 128), :] = (
                (au >> 16) | (bu & jnp.uint32(0xFFFF0000)))


def _transpose_tables(tables_t):
    grid = (NUM_ENT + TBLK - 1) // TBLK
    return pl.pallas_call(
        _transpose_block,
        grid=(grid,),
        in_specs=[pl.BlockSpec((D, TBLK), lambda i: (0, i))] * 4,
        out_specs=[pl.BlockSpec((TBLK, D // 2), lambda i: (i, 0))] * 4,
        out_shape=[jax.ShapeDtypeStruct((NUM_ENT, D // 2), jnp.uint32)] * 4,
        compiler_params=pltpu.CompilerParams(
            dimension_semantics=("parallel",)),
    )(*tables_t)


def _lane_perm(x, idx):
    dnums = lax.GatherDimensionNumbers(
        offset_dims=(), collapsed_slice_dims=(0,), start_index_map=(0,))
    return lax.gather(x, idx[:, None], dnums, slice_sizes=(1,),
                      mode=lax.GatherScatterMode.PROMISE_IN_BOUNDS)


def _body(heads_hbm, rels_hbm, tails_hbm, eh_hbm, et_hbm, r_hbm, ri_hbm,
          out_hbm, h_idx, r_idx, t_idx,
          bufs_a, bufs_b, out_v, sem_a, sem_b):
    cid = lax.axis_index("c")
    sid = lax.axis_index("s")
    wid = sid * NUM_CORES + cid
    base = wid * B_PER_W

    pltpu.sync_copy(heads_hbm.at[pl.ds(base, B_PER_W)], h_idx)
    pltpu.sync_copy(rels_hbm.at[pl.ds(base, B_PER_W)], r_idx)
    pltpu.sync_copy(tails_hbm.at[pl.ds(base, B_PER_W)], t_idx)

    iota = lax.iota(jnp.int32, 16)

    def issue(chunk, bufs, sem):
        hh_v, ht_v, th_v, tt_v, r_v, ri_v = bufs
        off = pl.multiple_of(chunk * W, W)
        hvec = h_idx[pl.ds(off, W)]
        rvec = r_idx[pl.ds(off, W)]
        tvec = t_idx[pl.ds(off, W)]
        for j in range(W):
            hv = hvec[j]
            rv = rvec[j]
            tv = tvec[j]
            pltpu.async_copy(eh_hbm.at[hv], hh_v.at[j], sem)
            pltpu.async_copy(eh_hbm.at[tv], ht_v.at[j], sem)
            pltpu.async_copy(et_hbm.at[hv], th_v.at[j], sem)
            pltpu.async_copy(et_hbm.at[tv], tt_v.at[j], sem)
            pltpu.async_copy(r_hbm.at[rv], r_v.at[j], sem)
            pltpu.async_copy(ri_hbm.at[rv], ri_v.at[j], sem)

    def drain(bufs, sem):
        for buf in bufs:
            pltpu.make_async_copy(eh_hbm.at[pl.ds(0, W)], buf, sem).wait()

    hi_mask = jnp.full((16,), 0xFFFF0000, jnp.uint32)

    def compute(chunk, bufs):
        acc = jnp.zeros((16,), jnp.float32)
        for j in range(W):
            s = None
            for q in range(2):
                sl = pl.ds(q * 16, 16)
                words = [buf[j, sl] for buf in bufs]
                for half in range(2):
                    if half == 0:
                        vals = [lax.bitcast_convert_type(w << 16, jnp.float32)
                                for w in words]
                    else:
                        vals = [lax.bitcast_convert_type(w & hi_mask,
                                                         jnp.float32)
                                for w in words]
                    hh, ht, th, tt, r, ri = vals
                    p = hh * r * tt + ht * ri * th
                    s = p if s is None else s + p
            for sh in (8, 4, 2, 1):
                s = s + _lane_perm(s, iota ^ sh)
            acc = jnp.where(iota == j, s, acc)
        acc = jnp.clip(acc * 0.5, -20.0, 20.0)
        out_v[pl.ds(pl.multiple_of(chunk * W, W), W)] = acc

    issue(0, bufs_a, sem_a)

    def pair_body(k, _):
        c0 = k * 2
        issue(c0 + 1, bufs_b, sem_b)
        drain(bufs_a, sem_a)
        compute(c0, bufs_a)

        @pl.when(c0 + 2 < NCHUNK)
        def _():
            issue(c0 + 2, bufs_a, sem_a)

        drain(bufs_b, sem_b)
        compute(c0 + 1, bufs_b)
        return 0

    lax.fori_loop(0, NCHUNK // 2, pair_body, 0)

    pltpu.sync_copy(out_v, out_hbm.at[pl.ds(base, B_PER_W)])


def kernel(batch, ent_h_embs, ent_t_embs, rel_embs, rel_inv_embs):
    mesh = plsc.VectorSubcoreMesh(core_axis_name="c", subcore_axis_name="s")
    k = functools.partial(
        pl.kernel,
        mesh=mesh,
        out_type=jax.ShapeDtypeStruct((BATCH,), jnp.float32),
        scratch_types=[
            pltpu.VMEM((B_PER_W,), jnp.int32),
            pltpu.VMEM((B_PER_W,), jnp.int32),
            pltpu.VMEM((B_PER_W,), jnp.int32),
            [pltpu.VMEM((W, D // 2), jnp.uint32) for _ in range(6)],
            [pltpu.VMEM((W, D // 2), jnp.uint32) for _ in range(6)],
            pltpu.VMEM((B_PER_W,), jnp.float32),
            pltpu.SemaphoreType.DMA,
            pltpu.SemaphoreType.DMA,
        ],
    )(_body)
    heads = batch[:, 0]
    rels = batch[:, 1]
    tails = batch[:, 2]
    eh, et, r, ri = _transpose_tables(
        (ent_h_embs.T, ent_t_embs.T, rel_embs.T, rel_inv_embs.T))
    return k(heads, rels, tails, eh, et, r, ri)
